# Initial kernel scaffold; baseline (speedup 1.0000x reference)
import jax
import jax.numpy as jnp
from jax.experimental import pallas as pl


def kernel(state, nodes, batch_id_nodes, W_attend, b_attend, W_align, b_align):
    raise NotImplementedError("write your pallas kernel here")


# speedup vs baseline: 5.6933x; 5.6933x over previous
import functools

import jax
import jax.numpy as jnp
from jax import lax
from jax.experimental import pallas as pl
from jax.experimental.pallas import tpu as pltpu
from jax.experimental.pallas import tpu_sc as plsc

_NEG_BIG = -3.4e38
_BLK = 2000


def _tc_body(nodes_ref, ids_ref, state_ref, wa_ref, w1_ref, w2_ref, ba_ref,
             bal_ref, z_ref, mm_ref, splus_ref, m_scr, s_scr):
    i = pl.program_id(0)
    nb = pl.num_programs(0)
    bsz = m_scr.shape[0]
    blk = nodes_ref.shape[0]

    @pl.when(i == 0)
    def _init():
        m_scr[...] = jnp.full(m_scr.shape, _NEG_BIG, jnp.float32)
        s_scr[...] = jnp.zeros(s_scr.shape, jnp.float32)
        mm_ref[...] = jnp.zeros(mm_ref.shape, jnp.float32)
        splus_ref[...] = lax.dot_general(
            state_ref[...], w1_ref[...], (((1,), (0,)), ((), ())),
            preferred_element_type=jnp.float32,
            precision=lax.Precision.HIGHEST) + bal_ref[0, 0]

    ids = ids_ref[0]
    nodes = nodes_ref[...]

    z = lax.dot_general(w2_ref[...], nodes, (((0,), (1,)), ((), ())),
                        preferred_element_type=jnp.float32,
                        precision=lax.Precision.HIGHEST)
    z_ref[0] = z

    onehot_t = (jnp.broadcast_to(ids, (bsz, blk)) ==
                lax.broadcasted_iota(jnp.int32, (bsz, blk), 0))

    m_loc = jnp.max(jnp.where(onehot_t, jnp.broadcast_to(z, (bsz, blk)),
                              _NEG_BIG), axis=1, keepdims=True)
    m_old = m_scr[...]
    m_new = jnp.maximum(m_old, m_loc)
    factor = jnp.exp(m_old - m_new)
    m_scr[...] = m_new

    oh_f = onehot_t.astype(jnp.float32)
    m_g = lax.dot_general(m_new, oh_f, (((0,), (0,)), ((), ())),
                          preferred_element_type=jnp.float32,
                          precision=lax.Precision.HIGHEST)
    p = jnp.exp(z - m_g)
    w_oh = oh_f * p

    s_scr[...] = s_scr[...] * factor + jnp.sum(w_oh, axis=1, keepdims=True)

    attend = lax.dot_general(nodes, wa_ref[...], (((1,), (0,)), ((), ())),
                             preferred_element_type=jnp.float32,
                             precision=lax.Precision.HIGHEST) + ba_ref[...]
    attend = jnp.where(attend > 0, attend, 0.2 * attend)
    mm_ref[...] = mm_ref[...] * factor + lax.dot_general(
        w_oh, attend, (((1,), (0,)), ((), ())),
        preferred_element_type=jnp.float32,
        precision=lax.Precision.HIGHEST)

    @pl.when(i == nb - 1)
    def _fin():
        mm = mm_ref[...] / jnp.maximum(s_scr[...], 1e-12)
        mm_ref[...] = jnp.where(mm > 0, mm, jnp.exp(mm) - 1.0)


def _sc_body(chunk, vregs, z_hbm, ids_hbm, splus_hbm, out_hbm,
             z_v, ids_v, sp_v, out_v):
    c = lax.axis_index("c")
    s = lax.axis_index("s")
    wid = s * 2 + c
    base = wid * chunk
    pltpu.sync_copy(z_hbm.at[pl.ds(base, chunk)], z_v)
    pltpu.sync_copy(ids_hbm.at[pl.ds(base, chunk)], ids_v)
    pltpu.sync_copy(splus_hbm, sp_v)

    def body(i, carry):
        off = i * 16
        idx = ids_v[pl.ds(off, 16)]
        g = plsc.load_gather(sp_v, [idx])
        out_v[pl.ds(off, 16)] = z_v[pl.ds(off, 16)] + g
        return carry

    lax.fori_loop(0, vregs, body, 0, unroll=4)
    pltpu.sync_copy(out_v, out_hbm.at[pl.ds(base, chunk)])


def _tc_stage(state, nodes, ids3, W_attend, w1, w2, ba2, bal2):
    n, f = nodes.shape
    bsz = state.shape[0]
    u = W_attend.shape[1]
    blk = _BLK
    nb = n // blk
    return pl.pallas_call(
        _tc_body,
        grid=(nb,),
        in_specs=[
            pl.BlockSpec((blk, f), lambda i: (i, 0)),
            pl.BlockSpec((1, 1, blk), lambda i: (i, 0, 0)),
            pl.BlockSpec((bsz, f), lambda i: (0, 0)),
            pl.BlockSpec((f, u), lambda i: (0, 0)),
            pl.BlockSpec((f, 1), lambda i: (0, 0)),
            pl.BlockSpec((f, 1), lambda i: (0, 0)),
            pl.BlockSpec((1, u), lambda i: (0, 0)),
            pl.BlockSpec((1, 1), lambda i: (0, 0)),
        ],
        out_specs=[
            pl.BlockSpec((1, 1, blk), lambda i: (i, 0, 0)),
            pl.BlockSpec((bsz, u), lambda i: (0, 0)),
            pl.BlockSpec((bsz, 1), lambda i: (0, 0)),
        ],
        out_shape=[
            jax.ShapeDtypeStruct((nb, 1, blk), jnp.float32),
            jax.ShapeDtypeStruct((bsz, u), jnp.float32),
            jax.ShapeDtypeStruct((bsz, 1), jnp.float32),
        ],
        scratch_shapes=[
            pltpu.VMEM((bsz, 1), jnp.float32),
            pltpu.VMEM((bsz, 1), jnp.float32),
        ],
    )(nodes, ids3, state, W_attend, w1, w2, ba2, bal2)


def _sc_stage(z_pad, ids_pad, splus_v, chunk, vregs):
    bsz = splus_v.shape[0]
    n_pad = z_pad.shape[0]
    sc_fn = pl.kernel(
        functools.partial(_sc_body, chunk, vregs),
        out_type=jax.ShapeDtypeStruct((n_pad,), jnp.float32),
        mesh=plsc.VectorSubcoreMesh(core_axis_name="c", subcore_axis_name="s",
                                    num_cores=2, num_subcores=16),
        compiler_params=pltpu.CompilerParams(needs_layout_passes=False),
        scratch_types=[
            pltpu.VMEM((chunk,), jnp.float32),
            pltpu.VMEM((chunk,), jnp.int32),
            pltpu.VMEM((bsz,), jnp.float32),
            pltpu.VMEM((chunk,), jnp.float32),
        ],
    )
    return sc_fn(z_pad, ids_pad, splus_v)


def kernel(state, nodes, batch_id_nodes, W_attend, b_attend, W_align, b_align):
    n, f = nodes.shape
    bsz = state.shape[0]
    u = W_attend.shape[1]
    blk = _BLK
    nb = n // blk

    ids32 = batch_id_nodes.astype(jnp.int32)
    ids3 = ids32.reshape(nb, 1, blk)
    w1 = W_align[:f]
    w2 = W_align[f:]
    ba2 = b_attend.reshape(1, u)
    bal2 = b_align.reshape(1, 1)

    z3, mm, splus = _tc_stage(state, nodes, ids3, W_attend, w1, w2, ba2, bal2)

    n_workers = 32
    chunk = -(-n // (n_workers * 16)) * 16
    chunk = -(-chunk // 8) * 8
    n_pad = chunk * n_workers
    vregs = chunk // 16

    z_pad = jnp.pad(z3.reshape(n), (0, n_pad - n))
    ids_pad = jnp.pad(ids32, (0, n_pad - n))

    align_pad = _sc_stage(z_pad, ids_pad, splus.reshape(bsz), chunk, vregs)
    align = align_pad[:n].reshape(n, 1)
    return (mm, align)


# speedup vs baseline: 11.4699x; 2.0147x over previous
import functools

import jax
import jax.numpy as jnp
from jax import lax
from jax.experimental import pallas as pl
from jax.experimental.pallas import tpu as pltpu
from jax.experimental.pallas import tpu_sc as plsc

_NEG_BIG = -3.38953138925153547590470800371487866880e+38
_BLK = 2000


def _tc_body(nodes_ref, ids_ref, state_ref, wa_ref, w1_ref, w2_ref, ba_ref,
             bal_ref, z_ref, mm_ref, splus_ref, m_scr, s_scr):
    i = pl.program_id(0)
    nb = pl.num_programs(0)
    bsz = m_scr.shape[0]
    blk = nodes_ref.shape[0]

    @pl.when(i == 0)
    def _init():
        m_scr[...] = jnp.full(m_scr.shape, _NEG_BIG, jnp.float32)
        s_scr[...] = jnp.zeros(s_scr.shape, jnp.float32)
        mm_ref[...] = jnp.zeros(mm_ref.shape, jnp.float32)
        splus_ref[...] = lax.dot_general(
            state_ref[...], w1_ref[...], (((1,), (0,)), ((), ())),
            preferred_element_type=jnp.float32,
            precision=lax.Precision.HIGHEST) + bal_ref[0, 0]

    ids = ids_ref[0]
    nodes = nodes_ref[...]

    z = lax.dot_general(w2_ref[...], nodes, (((0,), (1,)), ((), ())),
                        preferred_element_type=jnp.float32,
                        precision=lax.Precision.HIGHEST)
    z_ref[0] = z

    onehot_t = (jnp.broadcast_to(ids.astype(jnp.int16), (bsz, blk)) ==
                lax.broadcasted_iota(jnp.int16, (bsz, blk), 0))
    oh_bf = onehot_t.astype(jnp.bfloat16)

    m_loc = jnp.max(
        jnp.where(onehot_t,
                  jnp.broadcast_to(z.astype(jnp.bfloat16), (bsz, blk)),
                  jnp.bfloat16(jnp.finfo(jnp.bfloat16).min)),
        axis=1, keepdims=True).astype(jnp.float32)
    m_old = m_scr[...]
    m_new = jnp.maximum(m_old, m_loc)
    factor = jnp.exp(m_old - m_new)
    m_scr[...] = m_new

    m_g = lax.dot_general(m_new.astype(jnp.bfloat16), oh_bf,
                          (((0,), (0,)), ((), ())),
                          preferred_element_type=jnp.float32)
    p = jnp.exp(z - m_g)
    w_oh = oh_bf * p.astype(jnp.bfloat16)

    s_scr[...] = s_scr[...] * factor + lax.dot_general(
        w_oh, jnp.ones((blk, 1), jnp.bfloat16), (((1,), (0,)), ((), ())),
        preferred_element_type=jnp.float32)

    attend = lax.dot_general(nodes.astype(jnp.bfloat16), wa_ref[...],
                             (((1,), (0,)), ((), ())),
                             preferred_element_type=jnp.float32) + ba_ref[...]
    attend = jnp.where(attend > 0, attend, 0.2 * attend)
    mm_ref[...] = mm_ref[...] * factor + lax.dot_general(
        w_oh, attend.astype(jnp.bfloat16), (((1,), (0,)), ((), ())),
        preferred_element_type=jnp.float32)

    @pl.when(i == nb - 1)
    def _fin():
        mm = mm_ref[...] / jnp.maximum(s_scr[...], 1e-12)
        mm_ref[...] = jnp.where(mm > 0, mm, jnp.exp(mm) - 1.0)


def _sc_body(chunk, vregs, z_hbm, ids_hbm, splus_hbm, out_hbm,
             z_v, ids_v, sp_v, out_v):
    c = lax.axis_index("c")
    s = lax.axis_index("s")
    wid = s * 2 + c
    base = wid * chunk
    pltpu.sync_copy(z_hbm.at[pl.ds(base, chunk)], z_v)
    pltpu.sync_copy(ids_hbm.at[pl.ds(base, chunk)], ids_v)
    pltpu.sync_copy(splus_hbm, sp_v)

    def body(i, carry):
        off = i * 16
        idx = ids_v[pl.ds(off, 16)]
        g = plsc.load_gather(sp_v, [idx])
        out_v[pl.ds(off, 16)] = z_v[pl.ds(off, 16)] + g
        return carry

    lax.fori_loop(0, vregs, body, 0, unroll=4)
    pltpu.sync_copy(out_v, out_hbm.at[pl.ds(base, chunk)])


def _tc_stage(state, nodes, ids3, W_attend, w1, w2, ba2, bal2):
    n, f = nodes.shape
    bsz = state.shape[0]
    u = W_attend.shape[1]
    blk = _BLK
    nb = n // blk
    return pl.pallas_call(
        _tc_body,
        grid=(nb,),
        in_specs=[
            pl.BlockSpec((blk, f), lambda i: (i, 0)),
            pl.BlockSpec((1, 1, blk), lambda i: (i, 0, 0)),
            pl.BlockSpec((bsz, f), lambda i: (0, 0)),
            pl.BlockSpec((f, u), lambda i: (0, 0)),
            pl.BlockSpec((f, 1), lambda i: (0, 0)),
            pl.BlockSpec((f, 1), lambda i: (0, 0)),
            pl.BlockSpec((1, u), lambda i: (0, 0)),
            pl.BlockSpec((1, 1), lambda i: (0, 0)),
        ],
        out_specs=[
            pl.BlockSpec((1, 1, blk), lambda i: (i, 0, 0)),
            pl.BlockSpec((bsz, u), lambda i: (0, 0)),
            pl.BlockSpec((bsz, 1), lambda i: (0, 0)),
        ],
        out_shape=[
            jax.ShapeDtypeStruct((nb, 1, blk), jnp.float32),
            jax.ShapeDtypeStruct((bsz, u), jnp.float32),
            jax.ShapeDtypeStruct((bsz, 1), jnp.float32),
        ],
        scratch_shapes=[
            pltpu.VMEM((bsz, 1), jnp.float32),
            pltpu.VMEM((bsz, 1), jnp.float32),
        ],
    )(nodes, ids3, state, W_attend.astype(jnp.bfloat16), w1, w2, ba2, bal2)


def _sc_stage(z_pad, ids_pad, splus_v, chunk, vregs):
    bsz = splus_v.shape[0]
    n_pad = z_pad.shape[0]
    sc_fn = pl.kernel(
        functools.partial(_sc_body, chunk, vregs),
        out_type=jax.ShapeDtypeStruct((n_pad,), jnp.float32),
        mesh=plsc.VectorSubcoreMesh(core_axis_name="c", subcore_axis_name="s",
                                    num_cores=2, num_subcores=16),
        compiler_params=pltpu.CompilerParams(needs_layout_passes=False),
        scratch_types=[
            pltpu.VMEM((chunk,), jnp.float32),
            pltpu.VMEM((chunk,), jnp.int32),
            pltpu.VMEM((bsz,), jnp.float32),
            pltpu.VMEM((chunk,), jnp.float32),
        ],
    )
    return sc_fn(z_pad, ids_pad, splus_v)


def kernel(state, nodes, batch_id_nodes, W_attend, b_attend, W_align, b_align):
    n, f = nodes.shape
    bsz = state.shape[0]
    u = W_attend.shape[1]
    blk = _BLK
    nb = n // blk

    ids32 = batch_id_nodes.astype(jnp.int32)
    ids3 = ids32.reshape(nb, 1, blk)
    w1 = W_align[:f]
    w2 = W_align[f:]
    ba2 = b_attend.reshape(1, u)
    bal2 = b_align.reshape(1, 1)

    z3, mm, splus = _tc_stage(state, nodes, ids3, W_attend, w1, w2, ba2, bal2)

    n_workers = 32
    chunk = -(-n // (n_workers * 16)) * 16
    chunk = -(-chunk // 8) * 8
    n_pad = chunk * n_workers
    vregs = chunk // 16

    z_pad = jnp.pad(z3.reshape(n), (0, n_pad - n))
    ids_pad = jnp.pad(ids32, (0, n_pad - n))

    align_pad = _sc_stage(z_pad, ids_pad, splus.reshape(bsz), chunk, vregs)
    align = align_pad[:n].reshape(n, 1)
    return (mm, align)


# speedup vs baseline: 13.1952x; 1.1504x over previous
import functools

import jax
import jax.numpy as jnp
from jax import lax
from jax.experimental import pallas as pl
from jax.experimental.pallas import tpu as pltpu
from jax.experimental.pallas import tpu_sc as plsc

_NEG_BIG = -3.38953138925153547590470800371487866880e+38
_BLK = 2000


def _tc_body(nodes_ref, ids_ref, state_ref, wa_ref, w1_ref, w2_ref, ba_ref,
             bal_ref, z_ref, mm_ref, splus_ref, m_scr, s_scr):
    i = pl.program_id(0)
    nb = pl.num_programs(0)
    bsz = m_scr.shape[0]
    blk = nodes_ref.shape[0]

    @pl.when(i == 0)
    def _init():
        m_scr[...] = jnp.full(m_scr.shape, _NEG_BIG, jnp.float32)
        s_scr[...] = jnp.zeros(s_scr.shape, jnp.float32)
        mm_ref[...] = jnp.zeros(mm_ref.shape, jnp.float32)
        splus_ref[...] = lax.dot_general(
            state_ref[...], w1_ref[...], (((1,), (0,)), ((), ())),
            preferred_element_type=jnp.float32,
            precision=lax.Precision.HIGHEST) + bal_ref[0, 0]

    ids = ids_ref[0]
    nodes = nodes_ref[...]

    z = lax.dot_general(w2_ref[...], nodes, (((0,), (1,)), ((), ())),
                        preferred_element_type=jnp.float32,
                        precision=lax.Precision.HIGHEST)
    z_ref[0] = z

    onehot_t = (jnp.broadcast_to(ids.astype(jnp.int16), (bsz, blk)) ==
                lax.broadcasted_iota(jnp.int16, (bsz, blk), 0))

    m_blk = jnp.max(z, axis=1, keepdims=True)
    m_old = m_scr[...]
    m_new = jnp.maximum(m_old, m_blk)
    factor = jnp.exp(m_old - m_new)
    scale_b = jnp.exp(m_blk - m_new)
    m_scr[...] = m_new

    p_bf = jnp.exp(z - m_blk).astype(jnp.bfloat16)
    w_oh = jnp.where(onehot_t, jnp.broadcast_to(p_bf, (bsz, blk)),
                     jnp.bfloat16(0))

    s_scr[...] = s_scr[...] * factor + scale_b * lax.dot_general(
        w_oh, jnp.ones((blk, 1), jnp.bfloat16), (((1,), (0,)), ((), ())),
        preferred_element_type=jnp.float32)

    attend = lax.dot_general(nodes.astype(jnp.bfloat16), wa_ref[...],
                             (((1,), (0,)), ((), ())),
                             preferred_element_type=jnp.float32
                             ).astype(jnp.bfloat16) + ba_ref[...]
    attend = jnp.where(attend > 0, attend, jnp.bfloat16(0.2) * attend)
    mm_ref[...] = mm_ref[...] * factor + scale_b * lax.dot_general(
        w_oh, attend, (((1,), (0,)), ((), ())),
        preferred_element_type=jnp.float32)

    @pl.when(i == nb - 1)
    def _fin():
        mm = mm_ref[...] / jnp.maximum(s_scr[...], 1e-12)
        mm_ref[...] = jnp.where(mm > 0, mm, jnp.exp(mm) - 1.0)


def _sc_body(chunk, vregs, z_hbm, ids_hbm, splus_hbm, out_hbm,
             z_v, ids_v, sp_v, out_v):
    c = lax.axis_index("c")
    s = lax.axis_index("s")
    wid = s * 2 + c
    base = wid * chunk
    pltpu.sync_copy(z_hbm.at[pl.ds(base, chunk)], z_v)
    pltpu.sync_copy(ids_hbm.at[pl.ds(base, chunk)], ids_v)
    pltpu.sync_copy(splus_hbm, sp_v)

    def body(i, carry):
        off = i * 16
        idx = ids_v[pl.ds(off, 16)]
        g = plsc.load_gather(sp_v, [idx])
        out_v[pl.ds(off, 16)] = z_v[pl.ds(off, 16)] + g
        return carry

    lax.fori_loop(0, vregs, body, 0, unroll=4)
    pltpu.sync_copy(out_v, out_hbm.at[pl.ds(base, chunk)])


def _tc_stage(state, nodes, ids3, W_attend, w1, w2, ba2, bal2):
    n, f = nodes.shape
    bsz = state.shape[0]
    u = W_attend.shape[1]
    blk = _BLK
    nb = n // blk
    return pl.pallas_call(
        _tc_body,
        grid=(nb,),
        in_specs=[
            pl.BlockSpec((blk, f), lambda i: (i, 0)),
            pl.BlockSpec((1, 1, blk), lambda i: (i, 0, 0)),
            pl.BlockSpec((bsz, f), lambda i: (0, 0)),
            pl.BlockSpec((f, u), lambda i: (0, 0)),
            pl.BlockSpec((f, 1), lambda i: (0, 0)),
            pl.BlockSpec((f, 1), lambda i: (0, 0)),
            pl.BlockSpec((1, u), lambda i: (0, 0)),
            pl.BlockSpec((1, 1), lambda i: (0, 0)),
        ],
        out_specs=[
            pl.BlockSpec((1, 1, blk), lambda i: (i, 0, 0)),
            pl.BlockSpec((bsz, u), lambda i: (0, 0)),
            pl.BlockSpec((bsz, 1), lambda i: (0, 0)),
        ],
        out_shape=[
            jax.ShapeDtypeStruct((nb, 1, blk), jnp.float32),
            jax.ShapeDtypeStruct((bsz, u), jnp.float32),
            jax.ShapeDtypeStruct((bsz, 1), jnp.float32),
        ],
        scratch_shapes=[
            pltpu.VMEM((bsz, 1), jnp.float32),
            pltpu.VMEM((bsz, 1), jnp.float32),
        ],
    )(nodes, ids3, state, W_attend.astype(jnp.bfloat16), w1, w2, ba2, bal2)


def _sc_stage(z_pad, ids_pad, splus_v, chunk, vregs):
    bsz = splus_v.shape[0]
    n_pad = z_pad.shape[0]
    sc_fn = pl.kernel(
        functools.partial(_sc_body, chunk, vregs),
        out_type=jax.ShapeDtypeStruct((n_pad,), jnp.float32),
        mesh=plsc.VectorSubcoreMesh(core_axis_name="c", subcore_axis_name="s",
                                    num_cores=2, num_subcores=16),
        compiler_params=pltpu.CompilerParams(needs_layout_passes=False),
        scratch_types=[
            pltpu.VMEM((chunk,), jnp.float32),
            pltpu.VMEM((chunk,), jnp.int32),
            pltpu.VMEM((bsz,), jnp.float32),
            pltpu.VMEM((chunk,), jnp.float32),
        ],
    )
    return sc_fn(z_pad, ids_pad, splus_v)


def kernel(state, nodes, batch_id_nodes, W_attend, b_attend, W_align, b_align):
    n, f = nodes.shape
    bsz = state.shape[0]
    u = W_attend.shape[1]
    blk = _BLK
    nb = n // blk

    ids32 = batch_id_nodes.astype(jnp.int32)
    ids3 = ids32.reshape(nb, 1, blk)
    w1 = W_align[:f]
    w2 = W_align[f:]
    ba2 = b_attend.reshape(1, u).astype(jnp.bfloat16)
    bal2 = b_align.reshape(1, 1)

    z3, mm, splus = _tc_stage(state, nodes, ids3, W_attend, w1, w2, ba2, bal2)

    n_workers = 32
    chunk = -(-n // (n_workers * 16)) * 16
    chunk = -(-chunk // 8) * 8
    n_pad = chunk * n_workers
    vregs = chunk // 16

    z_pad = jnp.pad(z3.reshape(n), (0, n_pad - n))
    ids_pad = jnp.pad(ids32, (0, n_pad - n))

    align_pad = _sc_stage(z_pad, ids_pad, splus.reshape(bsz), chunk, vregs)
    align = align_pad[:n].reshape(n, 1)
    return (mm, align)


# speedup vs baseline: 15.8799x; 1.2035x over previous
import functools

import jax
import jax.numpy as jnp
from jax import lax
from jax.experimental import pallas as pl
from jax.experimental.pallas import tpu as pltpu
from jax.experimental.pallas import tpu_sc as plsc

_NEG_BIG = -3.38953138925153547590470800371487866880e+38
_BLK = 4000


def _tc_body(nodes_ref, ids_ref, state_ref, wa_ref, w1_ref, w2_ref, ba_ref,
             bal_ref, z_ref, mm_ref, splus_ref, m_scr, s_scr):
    i = pl.program_id(0)
    nb = pl.num_programs(0)
    bsz = m_scr.shape[0]
    blk = nodes_ref.shape[0]

    @pl.when(i == 0)
    def _init():
        m_scr[...] = jnp.full(m_scr.shape, _NEG_BIG, jnp.float32)
        s_scr[...] = jnp.zeros(s_scr.shape, jnp.float32)
        mm_ref[...] = jnp.zeros(mm_ref.shape, jnp.float32)
        splus_ref[...] = lax.dot_general(
            state_ref[...], w1_ref[...], (((1,), (0,)), ((), ())),
            preferred_element_type=jnp.float32,
            precision=lax.Precision.HIGHEST) + bal_ref[0, 0]

    ids = ids_ref[0]
    nodes = nodes_ref[...]

    z = lax.dot_general(w2_ref[...], nodes, (((0,), (1,)), ((), ())),
                        preferred_element_type=jnp.float32,
                        precision=lax.Precision.HIGHEST)
    z_ref[0] = z

    onehot_t = (jnp.broadcast_to(ids.astype(jnp.int16), (bsz, blk)) ==
                lax.broadcasted_iota(jnp.int16, (bsz, blk), 0))

    m_blk = jnp.max(z, axis=1, keepdims=True)
    m_old = m_scr[...]
    m_new = jnp.maximum(m_old, m_blk)
    factor = jnp.exp(m_old - m_new)
    scale_b = jnp.exp(m_blk - m_new)
    m_scr[...] = m_new

    p_bf = jnp.exp(z - m_blk).astype(jnp.bfloat16)
    w_oh = jnp.where(onehot_t, jnp.broadcast_to(p_bf, (bsz, blk)),
                     jnp.bfloat16(0))

    s_scr[...] = s_scr[...] * factor + scale_b * lax.dot_general(
        w_oh, jnp.ones((blk, 1), jnp.bfloat16), (((1,), (0,)), ((), ())),
        preferred_element_type=jnp.float32)

    attend = lax.dot_general(nodes.astype(jnp.bfloat16), wa_ref[...],
                             (((1,), (0,)), ((), ())),
                             preferred_element_type=jnp.float32
                             ).astype(jnp.bfloat16) + ba_ref[...]
    attend = jnp.where(attend > 0, attend, jnp.bfloat16(0.2) * attend)
    mm_ref[...] = mm_ref[...] * factor + scale_b * lax.dot_general(
        w_oh, attend, (((1,), (0,)), ((), ())),
        preferred_element_type=jnp.float32)

    @pl.when(i == nb - 1)
    def _fin():
        mm = mm_ref[...] / jnp.maximum(s_scr[...], 1e-12)
        mm_ref[...] = jnp.where(mm > 0, mm, jnp.exp(mm) - 1.0)


def _sc_body(chunk, vregs, z_hbm, ids_hbm, splus_hbm, out_hbm,
             z_v, ids_v, sp_v, out_v):
    c = lax.axis_index("c")
    s = lax.axis_index("s")
    wid = s * 2 + c
    base = wid * chunk
    pltpu.sync_copy(z_hbm.at[pl.ds(base, chunk)], z_v)
    pltpu.sync_copy(ids_hbm.at[pl.ds(base, chunk)], ids_v)
    pltpu.sync_copy(splus_hbm, sp_v)

    def body(i, carry):
        off = i * 16
        idx = ids_v[pl.ds(off, 16)]
        g = plsc.load_gather(sp_v, [idx])
        out_v[pl.ds(off, 16)] = z_v[pl.ds(off, 16)] + g
        return carry

    lax.fori_loop(0, vregs, body, 0, unroll=4)
    pltpu.sync_copy(out_v, out_hbm.at[pl.ds(base, chunk)])


def _tc_stage(state, nodes, ids3, W_attend, w1, w2, ba2, bal2):
    n, f = nodes.shape
    bsz = state.shape[0]
    u = W_attend.shape[1]
    blk = _BLK
    nb = n // blk
    return pl.pallas_call(
        _tc_body,
        grid=(nb,),
        in_specs=[
            pl.BlockSpec((blk, f), lambda i: (i, 0)),
            pl.BlockSpec((1, 1, blk), lambda i: (i, 0, 0)),
            pl.BlockSpec((bsz, f), lambda i: (0, 0)),
            pl.BlockSpec((f, u), lambda i: (0, 0)),
            pl.BlockSpec((f, 1), lambda i: (0, 0)),
            pl.BlockSpec((f, 1), lambda i: (0, 0)),
            pl.BlockSpec((1, u), lambda i: (0, 0)),
            pl.BlockSpec((1, 1), lambda i: (0, 0)),
        ],
        out_specs=[
            pl.BlockSpec((1, 1, blk), lambda i: (i, 0, 0)),
            pl.BlockSpec((bsz, u), lambda i: (0, 0)),
            pl.BlockSpec((bsz, 1), lambda i: (0, 0)),
        ],
        out_shape=[
            jax.ShapeDtypeStruct((nb, 1, blk), jnp.float32),
            jax.ShapeDtypeStruct((bsz, u), jnp.float32),
            jax.ShapeDtypeStruct((bsz, 1), jnp.float32),
        ],
        scratch_shapes=[
            pltpu.VMEM((bsz, 1), jnp.float32),
            pltpu.VMEM((bsz, 1), jnp.float32),
        ],
    )(nodes, ids3, state, W_attend.astype(jnp.bfloat16), w1, w2, ba2, bal2)


def _sc_stage(z_pad, ids_pad, splus_v, chunk, vregs):
    bsz = splus_v.shape[0]
    n_pad = z_pad.shape[0]
    sc_fn = pl.kernel(
        functools.partial(_sc_body, chunk, vregs),
        out_type=jax.ShapeDtypeStruct((n_pad,), jnp.float32),
        mesh=plsc.VectorSubcoreMesh(core_axis_name="c", subcore_axis_name="s",
                                    num_cores=2, num_subcores=16),
        compiler_params=pltpu.CompilerParams(needs_layout_passes=False),
        scratch_types=[
            pltpu.VMEM((chunk,), jnp.float32),
            pltpu.VMEM((chunk,), jnp.int32),
            pltpu.VMEM((bsz,), jnp.float32),
            pltpu.VMEM((chunk,), jnp.float32),
        ],
    )
    return sc_fn(z_pad, ids_pad, splus_v)


def kernel(state, nodes, batch_id_nodes, W_attend, b_attend, W_align, b_align):
    n, f = nodes.shape
    bsz = state.shape[0]
    u = W_attend.shape[1]
    blk = _BLK
    nb = n // blk

    ids32 = batch_id_nodes.astype(jnp.int32)
    ids3 = ids32.reshape(nb, 1, blk)
    w1 = W_align[:f]
    w2 = W_align[f:]
    ba2 = b_attend.reshape(1, u).astype(jnp.bfloat16)
    bal2 = b_align.reshape(1, 1)

    z3, mm, splus = _tc_stage(state, nodes, ids3, W_attend, w1, w2, ba2, bal2)

    n_workers = 32
    chunk = -(-n // (n_workers * 16)) * 16
    chunk = -(-chunk // 8) * 8
    n_pad = chunk * n_workers
    vregs = chunk // 16

    z_pad = jnp.pad(z3.reshape(n), (0, n_pad - n))
    ids_pad = jnp.pad(ids32, (0, n_pad - n))

    align_pad = _sc_stage(z_pad, ids_pad, splus.reshape(bsz), chunk, vregs)
    align = align_pad[:n].reshape(n, 1)
    return (mm, align)


# speedup vs baseline: 16.1037x; 1.0141x over previous
import functools

import jax
import jax.numpy as jnp
from jax import lax
from jax.experimental import pallas as pl
from jax.experimental.pallas import tpu as pltpu
from jax.experimental.pallas import tpu_sc as plsc

_NEG_BIG = -3.38953138925153547590470800371487866880e+38
_BLK = 10000


def _tc_body(nodes_ref, ids_ref, state_ref, wa_ref, w1_ref, w2_ref, ba_ref,
             bal_ref, z_ref, mm_ref, splus_ref, m_scr, s_scr):
    i = pl.program_id(0)
    nb = pl.num_programs(0)
    bsz = m_scr.shape[0]
    blk = nodes_ref.shape[0]

    @pl.when(i == 0)
    def _init():
        m_scr[...] = jnp.full(m_scr.shape, _NEG_BIG, jnp.float32)
        s_scr[...] = jnp.zeros(s_scr.shape, jnp.float32)
        mm_ref[...] = jnp.zeros(mm_ref.shape, jnp.float32)
        splus_ref[...] = lax.dot_general(
            state_ref[...], w1_ref[...], (((1,), (0,)), ((), ())),
            preferred_element_type=jnp.float32,
            precision=lax.Precision.HIGHEST) + bal_ref[0, 0]

    ids = ids_ref[0]
    nodes = nodes_ref[...]

    z = lax.dot_general(w2_ref[...], nodes, (((0,), (1,)), ((), ())),
                        preferred_element_type=jnp.float32,
                        precision=lax.Precision.HIGHEST)
    z_ref[0] = z

    onehot_t = (jnp.broadcast_to(ids.astype(jnp.int16), (bsz, blk)) ==
                lax.broadcasted_iota(jnp.int16, (bsz, blk), 0))

    m_blk = jnp.max(z, axis=1, keepdims=True)
    m_old = m_scr[...]
    m_new = jnp.maximum(m_old, m_blk)
    factor = jnp.exp(m_old - m_new)
    scale_b = jnp.exp(m_blk - m_new)
    m_scr[...] = m_new

    p_bf = jnp.exp(z - m_blk).astype(jnp.bfloat16)
    w_oh = jnp.where(onehot_t, jnp.broadcast_to(p_bf, (bsz, blk)),
                     jnp.bfloat16(0))

    s_scr[...] = s_scr[...] * factor + scale_b * lax.dot_general(
        w_oh, jnp.ones((blk, 1), jnp.bfloat16), (((1,), (0,)), ((), ())),
        preferred_element_type=jnp.float32)

    attend = lax.dot_general(nodes.astype(jnp.bfloat16), wa_ref[...],
                             (((1,), (0,)), ((), ())),
                             preferred_element_type=jnp.float32
                             ).astype(jnp.bfloat16) + ba_ref[...]
    attend = jnp.where(attend > 0, attend, jnp.bfloat16(0.2) * attend)
    mm_ref[...] = mm_ref[...] * factor + scale_b * lax.dot_general(
        w_oh, attend, (((1,), (0,)), ((), ())),
        preferred_element_type=jnp.float32)

    @pl.when(i == nb - 1)
    def _fin():
        mm = mm_ref[...] / jnp.maximum(s_scr[...], 1e-12)
        mm_ref[...] = jnp.where(mm > 0, mm, jnp.exp(mm) - 1.0)


def _sc_body(chunk, vregs, z_hbm, ids_hbm, splus_hbm, out_hbm,
             z_v, ids_v, sp_v, out_v):
    c = lax.axis_index("c")
    s = lax.axis_index("s")
    wid = s * 2 + c
    base = wid * chunk
    pltpu.sync_copy(z_hbm.at[pl.ds(base, chunk)], z_v)
    pltpu.sync_copy(ids_hbm.at[pl.ds(base, chunk)], ids_v)
    pltpu.sync_copy(splus_hbm, sp_v)

    def body(i, carry):
        off = i * 16
        idx = ids_v[pl.ds(off, 16)]
        g = plsc.load_gather(sp_v, [idx])
        out_v[pl.ds(off, 16)] = z_v[pl.ds(off, 16)] + g
        return carry

    lax.fori_loop(0, vregs, body, 0, unroll=4)
    pltpu.sync_copy(out_v, out_hbm.at[pl.ds(base, chunk)])


def _tc_stage(state, nodes, ids3, W_attend, w1, w2, ba2, bal2):
    n, f = nodes.shape
    bsz = state.shape[0]
    u = W_attend.shape[1]
    blk = _BLK
    nb = n // blk
    return pl.pallas_call(
        _tc_body,
        grid=(nb,),
        in_specs=[
            pl.BlockSpec((blk, f), lambda i: (i, 0)),
            pl.BlockSpec((1, 1, blk), lambda i: (i, 0, 0)),
            pl.BlockSpec((bsz, f), lambda i: (0, 0)),
            pl.BlockSpec((f, u), lambda i: (0, 0)),
            pl.BlockSpec((f, 1), lambda i: (0, 0)),
            pl.BlockSpec((f, 1), lambda i: (0, 0)),
            pl.BlockSpec((1, u), lambda i: (0, 0)),
            pl.BlockSpec((1, 1), lambda i: (0, 0)),
        ],
        out_specs=[
            pl.BlockSpec((1, 1, blk), lambda i: (i, 0, 0)),
            pl.BlockSpec((bsz, u), lambda i: (0, 0)),
            pl.BlockSpec((bsz, 1), lambda i: (0, 0)),
        ],
        out_shape=[
            jax.ShapeDtypeStruct((nb, 1, blk), jnp.float32),
            jax.ShapeDtypeStruct((bsz, u), jnp.float32),
            jax.ShapeDtypeStruct((bsz, 1), jnp.float32),
        ],
        scratch_shapes=[
            pltpu.VMEM((bsz, 1), jnp.float32),
            pltpu.VMEM((bsz, 1), jnp.float32),
        ],
    )(nodes, ids3, state, W_attend.astype(jnp.bfloat16), w1, w2, ba2, bal2)


def _sc_stage(z_pad, ids_pad, splus_v, chunk, vregs):
    bsz = splus_v.shape[0]
    n_pad = z_pad.shape[0]
    sc_fn = pl.kernel(
        functools.partial(_sc_body, chunk, vregs),
        out_type=jax.ShapeDtypeStruct((n_pad,), jnp.float32),
        mesh=plsc.VectorSubcoreMesh(core_axis_name="c", subcore_axis_name="s",
                                    num_cores=2, num_subcores=16),
        compiler_params=pltpu.CompilerParams(needs_layout_passes=False),
        scratch_types=[
            pltpu.VMEM((chunk,), jnp.float32),
            pltpu.VMEM((chunk,), jnp.int32),
            pltpu.VMEM((bsz,), jnp.float32),
            pltpu.VMEM((chunk,), jnp.float32),
        ],
    )
    return sc_fn(z_pad, ids_pad, splus_v)


def kernel(state, nodes, batch_id_nodes, W_attend, b_attend, W_align, b_align):
    n, f = nodes.shape
    bsz = state.shape[0]
    u = W_attend.shape[1]
    blk = _BLK
    nb = n // blk

    ids32 = batch_id_nodes.astype(jnp.int32)
    ids3 = ids32.reshape(nb, 1, blk)
    w1 = W_align[:f]
    w2 = W_align[f:]
    ba2 = b_attend.reshape(1, u).astype(jnp.bfloat16)
    bal2 = b_align.reshape(1, 1)

    z3, mm, splus = _tc_stage(state, nodes, ids3, W_attend, w1, w2, ba2, bal2)

    n_workers = 32
    chunk = -(-n // (n_workers * 16)) * 16
    chunk = -(-chunk // 8) * 8
    n_pad = chunk * n_workers
    vregs = chunk // 16

    z_pad = jnp.pad(z3.reshape(n), (0, n_pad - n))
    ids_pad = jnp.pad(ids32, (0, n_pad - n))

    align_pad = _sc_stage(z_pad, ids_pad, splus.reshape(bsz), chunk, vregs)
    align = align_pad[:n].reshape(n, 1)
    return (mm, align)


# speedup vs baseline: 21.9627x; 1.3638x over previous
import functools

import jax
import jax.numpy as jnp
from jax import lax
from jax.experimental import pallas as pl
from jax.experimental.pallas import tpu as pltpu
from jax.experimental.pallas import tpu_sc as plsc

_NEG_BIG = -3.38953138925153547590470800371487866880e+38
_BLK = 10000


def _tc_body(nodes_ref, ids_ref, state_ref, wa_ref, w1_ref, w2_ref, ba_ref,
             bal_ref, z_ref, mm_ref, splus_ref, m_scr, s_scr):
    i = pl.program_id(0)
    nb = pl.num_programs(0)
    bsz = m_scr.shape[0]
    blk = nodes_ref.shape[0]

    @pl.when(i == 0)
    def _init():
        m_scr[...] = jnp.full(m_scr.shape, _NEG_BIG, jnp.float32)
        s_scr[...] = jnp.zeros(s_scr.shape, jnp.float32)
        mm_ref[...] = jnp.zeros(mm_ref.shape, jnp.float32)
        splus_ref[...] = lax.dot_general(
            state_ref[...], w1_ref[...], (((1,), (0,)), ((), ())),
            preferred_element_type=jnp.float32,
            precision=lax.Precision.HIGHEST) + bal_ref[0, 0]

    ids = ids_ref[0]
    nodes_bf = nodes_ref[...].astype(jnp.bfloat16)

    z = lax.dot_general(w2_ref[...], nodes_bf, (((0,), (1,)), ((), ())),
                        preferred_element_type=jnp.float32)
    z_ref[0] = z

    onehot_t = (jnp.broadcast_to(ids.astype(jnp.int16), (bsz, blk)) ==
                lax.broadcasted_iota(jnp.int16, (bsz, blk), 0))

    m_blk = jnp.max(z, axis=1, keepdims=True)
    m_old = m_scr[...]
    m_new = jnp.maximum(m_old, m_blk)
    factor = jnp.exp(m_old - m_new)
    scale_b = jnp.exp(m_blk - m_new)
    m_scr[...] = m_new

    p_bf = jnp.exp(z - m_blk).astype(jnp.bfloat16)
    w_oh = jnp.where(onehot_t, jnp.broadcast_to(p_bf, (bsz, blk)),
                     jnp.bfloat16(0))

    s_scr[...] = s_scr[...] * factor + scale_b * lax.dot_general(
        w_oh, jnp.ones((blk, 1), jnp.bfloat16), (((1,), (0,)), ((), ())),
        preferred_element_type=jnp.float32)

    attend = lax.dot_general(nodes_bf, wa_ref[...],
                             (((1,), (0,)), ((), ())),
                             preferred_element_type=jnp.float32
                             ).astype(jnp.bfloat16) + ba_ref[...]
    attend = jnp.where(attend > 0, attend, jnp.bfloat16(0.2) * attend)
    mm_ref[...] = mm_ref[...] * factor + scale_b * lax.dot_general(
        w_oh, attend, (((1,), (0,)), ((), ())),
        preferred_element_type=jnp.float32)

    @pl.when(i == nb - 1)
    def _fin():
        mm = mm_ref[...] / jnp.maximum(s_scr[...], 1e-12)
        mm_ref[...] = jnp.where(mm > 0, mm, jnp.exp(mm) - 1.0)


def _sc_body(chunk, vregs, z_hbm, ids_hbm, splus_hbm, out_hbm,
             z_v, ids_v, sp_v, out_v):
    c = lax.axis_index("c")
    s = lax.axis_index("s")
    wid = s * 2 + c
    base = wid * chunk
    pltpu.sync_copy(z_hbm.at[pl.ds(base, chunk)], z_v)
    pltpu.sync_copy(ids_hbm.at[pl.ds(base, chunk)], ids_v)
    pltpu.sync_copy(splus_hbm, sp_v)

    def body(i, carry):
        off = i * 16
        idx = ids_v[pl.ds(off, 16)]
        g = plsc.load_gather(sp_v, [idx])
        out_v[pl.ds(off, 16)] = z_v[pl.ds(off, 16)] + g
        return carry

    lax.fori_loop(0, vregs, body, 0, unroll=4)
    pltpu.sync_copy(out_v, out_hbm.at[pl.ds(base, chunk)])


def _tc_stage(state, nodes, ids3, W_attend, w1, w2, ba2, bal2):
    n, f = nodes.shape
    bsz = state.shape[0]
    u = W_attend.shape[1]
    blk = _BLK
    nb = n // blk
    return pl.pallas_call(
        _tc_body,
        grid=(nb,),
        in_specs=[
            pl.BlockSpec((blk, f), lambda i: (i, 0)),
            pl.BlockSpec((1, 1, blk), lambda i: (i, 0, 0)),
            pl.BlockSpec((bsz, f), lambda i: (0, 0)),
            pl.BlockSpec((f, u), lambda i: (0, 0)),
            pl.BlockSpec((f, 1), lambda i: (0, 0)),
            pl.BlockSpec((f, 1), lambda i: (0, 0)),
            pl.BlockSpec((1, u), lambda i: (0, 0)),
            pl.BlockSpec((1, 1), lambda i: (0, 0)),
        ],
        out_specs=[
            pl.BlockSpec((1, 1, blk), lambda i: (i, 0, 0)),
            pl.BlockSpec((bsz, u), lambda i: (0, 0)),
            pl.BlockSpec((bsz, 1), lambda i: (0, 0)),
        ],
        out_shape=[
            jax.ShapeDtypeStruct((nb, 1, blk), jnp.float32),
            jax.ShapeDtypeStruct((bsz, u), jnp.float32),
            jax.ShapeDtypeStruct((bsz, 1), jnp.float32),
        ],
        scratch_shapes=[
            pltpu.VMEM((bsz, 1), jnp.float32),
            pltpu.VMEM((bsz, 1), jnp.float32),
        ],
    )(nodes, ids3, state, W_attend.astype(jnp.bfloat16), w1, w2, ba2, bal2)


def _sc_stage(z_pad, ids_pad, splus_v, chunk, vregs):
    bsz = splus_v.shape[0]
    n_pad = z_pad.shape[0]
    sc_fn = pl.kernel(
        functools.partial(_sc_body, chunk, vregs),
        out_type=jax.ShapeDtypeStruct((n_pad,), jnp.float32),
        mesh=plsc.VectorSubcoreMesh(core_axis_name="c", subcore_axis_name="s",
                                    num_cores=2, num_subcores=16),
        compiler_params=pltpu.CompilerParams(needs_layout_passes=False),
        scratch_types=[
            pltpu.VMEM((chunk,), jnp.float32),
            pltpu.VMEM((chunk,), jnp.int32),
            pltpu.VMEM((bsz,), jnp.float32),
            pltpu.VMEM((chunk,), jnp.float32),
        ],
    )
    return sc_fn(z_pad, ids_pad, splus_v)


def kernel(state, nodes, batch_id_nodes, W_attend, b_attend, W_align, b_align):
    n, f = nodes.shape
    bsz = state.shape[0]
    u = W_attend.shape[1]
    blk = _BLK
    nb = n // blk

    ids32 = batch_id_nodes.astype(jnp.int32)
    ids3 = ids32.reshape(nb, 1, blk)
    w1 = W_align[:f]
    w2 = W_align[f:].astype(jnp.bfloat16)
    ba2 = b_attend.reshape(1, u).astype(jnp.bfloat16)
    bal2 = b_align.reshape(1, 1)

    z3, mm, splus = _tc_stage(state, nodes, ids3, W_attend, w1, w2, ba2, bal2)

    n_workers = 32
    chunk = -(-n // (n_workers * 16)) * 16
    chunk = -(-chunk // 8) * 8
    n_pad = chunk * n_workers
    vregs = chunk // 16

    z_pad = jnp.pad(z3.reshape(n), (0, n_pad - n))
    ids_pad = jnp.pad(ids32, (0, n_pad - n))

    align_pad = _sc_stage(z_pad, ids_pad, splus.reshape(bsz), chunk, vregs)
    align = align_pad[:n].reshape(n, 1)
    return (mm, align)
